# baseline (speedup 1.0000x reference)
import functools

import jax
import jax.numpy as jnp
from jax.experimental import pallas as pl
from jax.experimental.pallas import tpu as pltpu


def _rup(x, m):
    return ((x + m - 1) // m) * m


_VMEM = 64 * 1024 * 1024


def _mm1_body(x_ref, w_ref, b_ref, *rest, relu, has_res):
    if has_res:
        r_ref, o_ref = rest
    else:
        (o_ref,) = rest
    y = jnp.dot(x_ref[...], w_ref[...], preferred_element_type=jnp.float32)
    y = y + b_ref[...]
    if has_res:
        y = y + r_ref[...].astype(jnp.float32)
    if relu:
        y = jnp.maximum(y, 0.0)
    o_ref[...] = y.astype(o_ref.dtype)


def _mmk_body(x_ref, w_ref, b_ref, *rest, nk, relu, has_res):
    if has_res:
        r_ref, o_ref, acc_ref = rest
    else:
        o_ref, acc_ref = rest
    kk = pl.program_id(2)

    @pl.when(kk == 0)
    def _():
        acc_ref[...] = jnp.zeros_like(acc_ref)

    acc_ref[...] += jnp.dot(x_ref[...], w_ref[...],
                            preferred_element_type=jnp.float32)

    @pl.when(kk == nk - 1)
    def _():
        y = acc_ref[...] + b_ref[...]
        if has_res:
            y = y + r_ref[...].astype(jnp.float32)
        if relu:
            y = jnp.maximum(y, 0.0)
        o_ref[...] = y.astype(o_ref.dtype)


def _pick_tm(m):
    for t in (512, 256, 128, 112, 64, 56, 32, 16, 8):
        if m % t == 0:
            return t
    return m


def _pick_tk(k):
    if k <= 1024:
        return k
    for t in (1024, 768, 512, 384, 256, 128):
        if k % t == 0:
            return t
    return k


@functools.lru_cache(maxsize=None)
def _mm_build(m, k, n, relu, has_res):
    tm, tk = _pick_tm(m), _pick_tk(k)
    tn = n if n <= 512 else (512 if n % 512 == 0 else n)
    nk = k // tk
    flops = 2 * m * k * n
    ba = (m * k + k * n) * 2 + m * n * 2 + 4 * n + (m * n * 2 if has_res else 0)
    if nk == 1:
        in_specs = [
            pl.BlockSpec((tm, k), lambda i, j: (i, 0)),
            pl.BlockSpec((k, tn), lambda i, j: (0, j)),
            pl.BlockSpec((1, tn), lambda i, j: (0, j)),
        ]
        if has_res:
            in_specs.append(pl.BlockSpec((tm, tn), lambda i, j: (i, j)))
        return pl.pallas_call(
            functools.partial(_mm1_body, relu=relu, has_res=has_res),
            out_shape=jax.ShapeDtypeStruct((m, n), jnp.bfloat16),
            grid=(m // tm, n // tn),
            in_specs=in_specs,
            out_specs=pl.BlockSpec((tm, tn), lambda i, j: (i, j)),
            compiler_params=pltpu.CompilerParams(
                dimension_semantics=("parallel", "parallel"),
                vmem_limit_bytes=_VMEM),
            cost_estimate=pl.CostEstimate(flops=flops, transcendentals=0,
                                          bytes_accessed=ba),
        )
    in_specs = [
        pl.BlockSpec((tm, tk), lambda i, j, kk: (i, kk)),
        pl.BlockSpec((tk, tn), lambda i, j, kk: (kk, j)),
        pl.BlockSpec((1, tn), lambda i, j, kk: (0, j)),
    ]
    if has_res:
        in_specs.append(pl.BlockSpec((tm, tn), lambda i, j, kk: (i, j)))
    return pl.pallas_call(
        functools.partial(_mmk_body, nk=nk, relu=relu, has_res=has_res),
        out_shape=jax.ShapeDtypeStruct((m, n), jnp.bfloat16),
        grid_spec=pltpu.PrefetchScalarGridSpec(
            num_scalar_prefetch=0,
            grid=(m // tm, n // tn, nk),
            in_specs=in_specs,
            out_specs=pl.BlockSpec((tm, tn), lambda i, j, kk: (i, j)),
            scratch_shapes=[pltpu.VMEM((tm, tn), jnp.float32)],
        ),
        compiler_params=pltpu.CompilerParams(
            dimension_semantics=("parallel", "parallel", "arbitrary"),
            vmem_limit_bytes=_VMEM),
        cost_estimate=pl.CostEstimate(flops=flops, transcendentals=0,
                                      bytes_accessed=ba),
    )


def _mm(x, w, bias, *, relu, res=None):
    m, k = x.shape
    n = w.shape[1]
    args = [x.astype(jnp.bfloat16), w, bias]
    if res is not None:
        args.append(res.astype(jnp.bfloat16))
    return _mm_build(m, k, n, bool(relu), res is not None)(*args)


def _c3_body(x_ref, w_ref, s_ref, o_ref, xcat_ref, *, bsz, h, w, c, hp, wpp):
    for dj in range(3):
        xcat_ref[:, 0:w, dj * c:(dj + 1) * c] = x_ref[0, :, dj:dj + w, :]
    rows = bsz * hp
    lout = rows - 2
    acc = None
    for di in range(3):
        a = xcat_ref[di:di + lout].reshape(lout * wpp, 3 * c)
        z = jnp.dot(a, w_ref[di], preferred_element_type=jnp.float32)
        acc = z if acc is None else acc + z
    y = acc + s_ref[...]
    y = jnp.maximum(y, 0.0).astype(o_ref.dtype)
    y = y.reshape(lout, wpp, o_ref.shape[-1])
    for b in range(bsz):
        o_ref[0, b * h:(b + 1) * h] = y[b * hp: b * hp + h, 0:w]


@functools.lru_cache(maxsize=None)
def _c3_build(g, bsz, h, w, c, cout, hp, wpp):
    rows = bsz * hp
    lout = rows - 2
    flops = 2 * g * lout * wpp * 3 * c * cout * 3
    ba = (g * rows * wpp * c + 9 * c * cout + g * bsz * h * w * cout) * 2
    return pl.pallas_call(
        functools.partial(_c3_body, bsz=bsz, h=h, w=w, c=c, hp=hp, wpp=wpp),
        out_shape=jax.ShapeDtypeStruct((g, bsz * h, w, cout), jnp.bfloat16),
        grid_spec=pltpu.PrefetchScalarGridSpec(
            num_scalar_prefetch=0,
            grid=(g,),
            in_specs=[
                pl.BlockSpec((1, rows, wpp, c), lambda i: (i, 0, 0, 0)),
                pl.BlockSpec((3, 3 * c, cout), lambda i: (0, 0, 0)),
                pl.BlockSpec((1, cout), lambda i: (0, 0)),
            ],
            out_specs=pl.BlockSpec((1, bsz * h, w, cout), lambda i: (i, 0, 0, 0)),
            scratch_shapes=[pltpu.VMEM((rows, wpp, 3 * c), jnp.bfloat16)],
        ),
        compiler_params=pltpu.CompilerParams(
            dimension_semantics=("parallel",),
            vmem_limit_bytes=_VMEM),
        cost_estimate=pl.CostEstimate(flops=flops, transcendentals=0,
                                      bytes_accessed=ba),
    )


def _conv3_s1(x, wfull, shift):
    n, h, w, c = x.shape
    cout = wfull.shape[1]
    w3 = wfull[:9 * c].reshape(3, 3 * c, cout)
    g = {64: 8, 128: 8, 256: 4, 512: 2}.get(c, 8)
    bsz = n // g
    hp = h + 2
    wpp = _rup(w + 2, 16)
    xp = jnp.pad(x, ((0, 0), (1, 1), (1, wpp - w - 1), (0, 0)))
    xp = xp.reshape(g, bsz * hp, wpp, c)
    out = _c3_build(g, bsz, h, w, c, cout, hp, wpp)(xp, w3, shift)
    return out.reshape(n, h, w, cout)


def _c1_body(x_ref, w_ref, s_ref, o_ref, cat_ref, *, ho, wo):
    v = x_ref[0]
    for s in range(4):
        cat_ref[:, :, s * 12:(s + 1) * 12] = v[:, s:s + 112, :]
    acc = None
    for di in range(7):
        off = di // 2
        a = cat_ref[off:off + 112].reshape(112 * 112, 48)
        z = jnp.dot(a, w_ref[di], preferred_element_type=jnp.float32)
        acc = z if acc is None else acc + z
    y = jnp.maximum(acc + s_ref[...], 0.0).astype(o_ref.dtype)
    y = y.reshape(112, 112, 64)
    yr = y.reshape(112, wo, 2, 64)
    a0, a1 = yr[:, :, 0], yr[:, :, 1]
    ninf = jnp.full((112, 1, 64), -jnp.inf, y.dtype)
    cw = jnp.maximum(jnp.maximum(a0, a1),
                     jnp.concatenate([ninf, a1[:, :-1]], axis=1))
    cr = cw.reshape(ho, 2, wo, 64)
    c0, c1 = cr[:, 0], cr[:, 1]
    ninf2 = jnp.full((1, wo, 64), -jnp.inf, y.dtype)
    o_ref[0] = jnp.maximum(jnp.maximum(c0, c1),
                           jnp.concatenate([ninf2, c1[:-1]], axis=0))


@functools.lru_cache(maxsize=None)
def _c1_build(n):
    return pl.pallas_call(
        functools.partial(_c1_body, ho=56, wo=56),
        out_shape=jax.ShapeDtypeStruct((n, 56, 56, 64), jnp.bfloat16),
        grid_spec=pltpu.PrefetchScalarGridSpec(
            num_scalar_prefetch=0,
            grid=(n,),
            in_specs=[
                pl.BlockSpec((1, 115, 116, 12), lambda i: (i, 0, 0, 0)),
                pl.BlockSpec((7, 48, 64), lambda i: (0, 0, 0)),
                pl.BlockSpec((1, 64), lambda i: (0, 0)),
            ],
            out_specs=pl.BlockSpec((1, 56, 56, 64), lambda i: (i, 0, 0, 0)),
            scratch_shapes=[pltpu.VMEM((115, 112, 48), jnp.bfloat16)],
        ),
        compiler_params=pltpu.CompilerParams(
            dimension_semantics=("parallel",),
            vmem_limit_bytes=_VMEM),
        cost_estimate=pl.CostEstimate(
            flops=2 * n * 112 * 112 * 24 * 64 * 7, transcendentals=0,
            bytes_accessed=n * (230 * 116 * 6 + 56 * 56 * 64) * 2),
    )


def _conv1_pool(x_nchw, wfull, shift):
    n = x_nchw.shape[0]
    t = jnp.transpose(x_nchw, (0, 2, 3, 1)).astype(jnp.bfloat16)
    tp = jnp.pad(t, ((0, 0), (3, 3), (3, 5), (0, 0)))
    xq = tp.reshape(n, 115, 2, 116, 2, 3).transpose(0, 1, 3, 2, 4, 5)
    xq = xq.reshape(n, 115, 116, 12)
    w7 = jnp.zeros((7, 48, 64), jnp.bfloat16)
    for di in range(7):
        er = di % 2
        for s in range(4):
            for ec in range(2):
                dj = 2 * s + ec
                if dj < 7:
                    blk = wfull[di * 21 + dj * 3: di * 21 + dj * 3 + 3, :64]
                    base = s * 12 + er * 6 + ec * 3
                    w7 = w7.at[di, base: base + 3, :].set(blk)
    return _c1_build(n)(xq, w7, shift)


def _c3s2_body(x_ref, w_ref, s_ref, o_ref, ecat_ref, ocat_ref,
               *, bsz, ho, wo, c, pairs, wpr, woc):
    rows = bsz * 2 * pairs
    v = x_ref[0].reshape(bsz * pairs, 2, wpr, 2 * c)
    ev, od = v[:, 0], v[:, 1]
    ecat_ref[:, 0:wpr, 0:2 * c] = ev
    ocat_ref[:, 0:wpr, 0:2 * c] = od
    ecat_ref[:, 0:wpr - 1, 2 * c:3 * c] = ev[:, 1:wpr, 0:c]
    ocat_ref[:, 0:wpr - 1, 2 * c:3 * c] = od[:, 1:wpr, 0:c]
    lout = bsz * pairs - 1
    acc = None
    for di in range(3):
        src = ecat_ref if di % 2 == 0 else ocat_ref
        a = src[di // 2:di // 2 + lout].reshape(lout * woc, 3 * c)
        z = jnp.dot(a, w_ref[di], preferred_element_type=jnp.float32)
        acc = z if acc is None else acc + z
    y = jnp.maximum(acc + s_ref[...], 0.0).astype(o_ref.dtype)
    y = y.reshape(lout, woc, o_ref.shape[-1])
    for b in range(bsz):
        o_ref[0, b * ho:(b + 1) * ho] = y[b * pairs: b * pairs + ho, 0:wo]


@functools.lru_cache(maxsize=None)
def _c3s2_build(g, bsz, ho, wo, c, cout, pairs, wpr, woc):
    rows = bsz * 2 * pairs
    lout = bsz * pairs - 1
    return pl.pallas_call(
        functools.partial(_c3s2_body, bsz=bsz, ho=ho, wo=wo, c=c,
                          pairs=pairs, wpr=wpr, woc=woc),
        out_shape=jax.ShapeDtypeStruct((g, bsz * ho, wo, cout), jnp.bfloat16),
        grid_spec=pltpu.PrefetchScalarGridSpec(
            num_scalar_prefetch=0,
            grid=(g,),
            in_specs=[
                pl.BlockSpec((1, rows, wpr, 2 * c), lambda i: (i, 0, 0, 0)),
                pl.BlockSpec((3, 3 * c, cout), lambda i: (0, 0, 0)),
                pl.BlockSpec((1, cout), lambda i: (0, 0)),
            ],
            out_specs=pl.BlockSpec((1, bsz * ho, wo, cout), lambda i: (i, 0, 0, 0)),
            scratch_shapes=[pltpu.VMEM((bsz * pairs, woc, 3 * c), jnp.bfloat16),
                            pltpu.VMEM((bsz * pairs, woc, 3 * c), jnp.bfloat16)],
        ),
        compiler_params=pltpu.CompilerParams(
            dimension_semantics=("parallel",),
            vmem_limit_bytes=_VMEM),
        cost_estimate=pl.CostEstimate(
            flops=2 * g * lout * woc * 3 * c * cout * 3, transcendentals=0,
            bytes_accessed=(g * rows * wpr * 2 * c + 9 * c * cout
                            + g * bsz * ho * wo * cout) * 2),
    )


def _conv3_s2(x, wfull, shift):
    n, h, w, c = x.shape
    cout = wfull.shape[1]
    ho, wo = h // 2, w // 2
    pairs = (h + 2) // 2
    wpr = (w + 2 + 1) // 2
    woc = _rup(wo, 16)
    w3 = wfull[:9 * c].reshape(3, 3 * c, cout)
    g = {128: 8, 256: 4, 512: 2}.get(c, 4)
    bsz = n // g
    xp = jnp.pad(x, ((0, 0), (1, 1), (1, 2 * wpr - w - 1), (0, 0)))
    xp = xp.reshape(g, bsz * 2 * pairs, wpr, 2 * c)
    out = _c3s2_build(g, bsz, ho, wo, c, cout, pairs, wpr, woc)(xp, w3, shift)
    return out.reshape(n, ho, wo, cout)


def _ds_body(x_ref, w_ref, s_ref, o_ref, *, ho, wo, c):
    v = x_ref[0].reshape(ho, 2, wo, 2 * c)[:, 0]
    a = v.reshape(ho * wo, 2 * c)
    z = jnp.dot(a, w_ref[...], preferred_element_type=jnp.float32)
    z = z + s_ref[...]
    o_ref[0] = z.astype(o_ref.dtype).reshape(ho, wo, o_ref.shape[-1])


@functools.lru_cache(maxsize=None)
def _ds_build(n, h, wo, c, cout):
    return pl.pallas_call(
        functools.partial(_ds_body, ho=h // 2, wo=wo, c=c),
        out_shape=jax.ShapeDtypeStruct((n, h // 2, wo, cout), jnp.bfloat16),
        grid_spec=pltpu.PrefetchScalarGridSpec(
            num_scalar_prefetch=0,
            grid=(n,),
            in_specs=[
                pl.BlockSpec((1, h, wo, 2 * c), lambda i: (i, 0, 0, 0)),
                pl.BlockSpec((2 * c, cout), lambda i: (0, 0)),
                pl.BlockSpec((1, cout), lambda i: (0, 0)),
            ],
            out_specs=pl.BlockSpec((1, h // 2, wo, cout), lambda i: (i, 0, 0, 0)),
        ),
        compiler_params=pltpu.CompilerParams(
            dimension_semantics=("parallel",),
            vmem_limit_bytes=_VMEM),
        cost_estimate=pl.CostEstimate(
            flops=2 * n * (h // 2) * wo * 2 * c * cout, transcendentals=0,
            bytes_accessed=(n * h * wo * c * 2 + 2 * c * cout
                            + n * (h // 2) * wo * cout) * 2),
    )


def _downsample(x, wfull, shift):
    n, h, w, c = x.shape
    cout = wfull.shape[1]
    wz = jnp.concatenate([wfull, jnp.zeros_like(wfull)], axis=0)
    xv = x.reshape(n, h, w // 2, 2 * c)
    return _ds_build(n, h, w // 2, c, cout)(xv, wz, shift)


def _pool_body(ee_ref, eo_ref, oe_ref, oo_ref, o_ref, *, ho, wo):
    ee, eo, oe, oo = ee_ref[0], eo_ref[0], oe_ref[0], oo_ref[0]
    a = jnp.maximum(jnp.maximum(ee[:, :wo], eo[:, :wo]), ee[:, 1:wo + 1])
    b = jnp.maximum(jnp.maximum(oe[:, :wo], oo[:, :wo]), oe[:, 1:wo + 1])
    o_ref[0] = jnp.maximum(jnp.maximum(a[:ho], b[:ho]), a[1:ho + 1])


@functools.lru_cache(maxsize=None)
def _pool_build(n, he, we, ho, wo, c):
    spec = pl.BlockSpec((1, he, we, c), lambda i: (i, 0, 0, 0))
    return pl.pallas_call(
        functools.partial(_pool_body, ho=ho, wo=wo),
        out_shape=jax.ShapeDtypeStruct((n, ho, wo, c), jnp.bfloat16),
        grid=(n,),
        in_specs=[spec, spec, spec, spec],
        out_specs=pl.BlockSpec((1, ho, wo, c), lambda i: (i, 0, 0, 0)),
        compiler_params=pltpu.CompilerParams(
            dimension_semantics=("parallel",),
            vmem_limit_bytes=_VMEM),
    )


def _maxpool(x):
    n, h, w, c = x.shape
    ho, wo = (h - 1) // 2 + 1, (w - 1) // 2 + 1
    he, we = ho + 1, wo + 1
    xp = jnp.pad(x, ((0, 0), (1, 2 * he - h - 1), (1, 2 * we - w - 1), (0, 0)),
                 constant_values=-jnp.inf)
    ee = xp[:, 0::2, 0::2]
    eo = xp[:, 0::2, 1::2]
    oe = xp[:, 1::2, 0::2]
    oo = xp[:, 1::2, 1::2]
    return _pool_build(n, he, we, ho, wo, c)(ee, eo, oe, oo)


def _tail_body(x_ref, fw_ref, fb_ref, w1_ref, b1_ref, w2_ref, b2_ref,
               w3c_ref, b3c_ref, w3r_ref, b3r_ref, c_ref, r_ref, *, inv_hw, hm):
    x = x_ref[...].astype(jnp.float32)
    pooled = jnp.sum(x, axis=1) * inv_hw
    feats = jnp.dot(pooled.astype(jnp.bfloat16), fw_ref[...],
                    preferred_element_type=jnp.float32) + fb_ref[...]
    h1 = jnp.dot(feats.astype(jnp.bfloat16), w1_ref[...],
                 preferred_element_type=jnp.float32) + b1_ref[...]
    h1 = jnp.maximum(h1, 0.0)
    h2 = jnp.dot(h1.astype(jnp.bfloat16), w2_ref[...],
                 preferred_element_type=jnp.float32) + b2_ref[...]
    h2 = jnp.maximum(h2, 0.0).astype(jnp.bfloat16)
    c_ref[...] = jnp.dot(h2[:, :hm], w3c_ref[...],
                         preferred_element_type=jnp.float32) + b3c_ref[...]
    r_ref[...] = jnp.dot(h2[:, hm:], w3r_ref[...],
                         preferred_element_type=jnp.float32) + b3r_ref[...]


@functools.lru_cache(maxsize=None)
def _tail_build(batch, hw, nc, nr):
    vmem = lambda: pl.BlockSpec(memory_space=pltpu.MemorySpace.VMEM)
    return pl.pallas_call(
        functools.partial(_tail_body, inv_hw=1.0 / hw, hm=32),
        out_shape=(jax.ShapeDtypeStruct((batch, nc), jnp.float32),
                   jax.ShapeDtypeStruct((batch, nr), jnp.float32)),
        in_specs=[vmem() for _ in range(11)],
        out_specs=(vmem(), vmem()),
        compiler_params=pltpu.CompilerParams(vmem_limit_bytes=_VMEM),
    )


def _tail(x, A):
    n, h, w, c = x.shape
    xr = x.reshape(n, h * w, c)
    cw1, cb1 = A["classify_fc1_w"], A["classify_fc1_b"]
    rw1, rb1 = A["regression_fc1_w"], A["regression_fc1_b"]
    cw2, cb2 = A["classify_fc2_w"], A["classify_fc2_b"]
    rw2, rb2 = A["regression_fc2_w"], A["regression_fc2_b"]
    d1 = cw1.shape[1]
    w1 = jnp.concatenate([cw1, rw1], axis=1)
    b1 = jnp.concatenate([cb1, rb1], axis=1)
    z = jnp.zeros_like(cw2)
    w2 = jnp.concatenate(
        [jnp.concatenate([cw2, z], axis=1), jnp.concatenate([z, rw2], axis=1)],
        axis=0)
    b2 = jnp.concatenate([cb2, rb2], axis=1)
    nc = A["classify_fc3_w"].shape[1]
    nr = A["regression_fc3_w"].shape[1]
    return _tail_build(n, h * w, nc, nr)(
        xr, A["fc_w"], A["fc_b"], w1, b1, w2, b2,
        A["classify_fc3_w"], A["classify_fc3_b"],
        A["regression_fc3_w"], A["regression_fc3_b"])


_ARCH = [(64, 3, 1), (128, 4, 2), (256, 6, 2), (512, 3, 2)]


def _forward(A):
    n = A["x"].shape[0]
    x = _conv1_pool(A["x"], A["conv1_w"], A["conv1_shift"])
    h = w = x.shape[1]
    cin = x.shape[3]

    for li, (planes, nblocks, lstride) in enumerate(_ARCH):
        for bi in range(nblocks):
            s = lstride if bi == 0 else 1
            pre = "layer%d_block%d_" % (li, bi)
            hn, wn = h // s, w // s
            if bi == 0:
                if s == 2:
                    idm = _downsample(x, A[pre + "downsample_w"],
                                      A[pre + "downsample_shift"])
                    idm = idm.reshape(n * hn * wn, -1)
                else:
                    idm = _mm(x.reshape(n * h * w, cin),
                              A[pre + "downsample_w"],
                              A[pre + "downsample_shift"], relu=False)
            else:
                idm = x.reshape(n * h * w, cin)
            y = _mm(x.reshape(n * h * w, cin),
                    A[pre + "conv1_w"], A[pre + "conv1_shift"], relu=True)
            y = y.reshape(n, h, w, planes)
            if s == 1:
                y = _conv3_s1(y, A[pre + "conv2_w"], A[pre + "conv2_shift"])
            else:
                y = _conv3_s2(y, A[pre + "conv2_w"], A[pre + "conv2_shift"])
            y = _mm(y.reshape(n * hn * wn, planes),
                    A[pre + "conv3_w"], A[pre + "conv3_shift"],
                    relu=True, res=idm)
            cin = 4 * planes
            h, w = hn, wn
            x = y.reshape(n, h, w, cin)

    return _tail(x, A)


def kernel(
    x,
    conv1_w, conv1_shift,
    layer0_block0_conv1_w, layer0_block0_conv1_shift,
    layer0_block0_conv2_w, layer0_block0_conv2_shift,
    layer0_block0_conv3_w, layer0_block0_conv3_shift,
    layer0_block0_downsample_w, layer0_block0_downsample_shift,
    layer0_block1_conv1_w, layer0_block1_conv1_shift,
    layer0_block1_conv2_w, layer0_block1_conv2_shift,
    layer0_block1_conv3_w, layer0_block1_conv3_shift,
    layer0_block2_conv1_w, layer0_block2_conv1_shift,
    layer0_block2_conv2_w, layer0_block2_conv2_shift,
    layer0_block2_conv3_w, layer0_block2_conv3_shift,
    layer1_block0_conv1_w, layer1_block0_conv1_shift,
    layer1_block0_conv2_w, layer1_block0_conv2_shift,
    layer1_block0_conv3_w, layer1_block0_conv3_shift,
    layer1_block0_downsample_w, layer1_block0_downsample_shift,
    layer1_block1_conv1_w, layer1_block1_conv1_shift,
    layer1_block1_conv2_w, layer1_block1_conv2_shift,
    layer1_block1_conv3_w, layer1_block1_conv3_shift,
    layer1_block2_conv1_w, layer1_block2_conv1_shift,
    layer1_block2_conv2_w, layer1_block2_conv2_shift,
    layer1_block2_conv3_w, layer1_block2_conv3_shift,
    layer1_block3_conv1_w, layer1_block3_conv1_shift,
    layer1_block3_conv2_w, layer1_block3_conv2_shift,
    layer1_block3_conv3_w, layer1_block3_conv3_shift,
    layer2_block0_conv1_w, layer2_block0_conv1_shift,
    layer2_block0_conv2_w, layer2_block0_conv2_shift,
    layer2_block0_conv3_w, layer2_block0_conv3_shift,
    layer2_block0_downsample_w, layer2_block0_downsample_shift,
    layer2_block1_conv1_w, layer2_block1_conv1_shift,
    layer2_block1_conv2_w, layer2_block1_conv2_shift,
    layer2_block1_conv3_w, layer2_block1_conv3_shift,
    layer2_block2_conv1_w, layer2_block2_conv1_shift,
    layer2_block2_conv2_w, layer2_block2_conv2_shift,
    layer2_block2_conv3_w, layer2_block2_conv3_shift,
    layer2_block3_conv1_w, layer2_block3_conv1_shift,
    layer2_block3_conv2_w, layer2_block3_conv2_shift,
    layer2_block3_conv3_w, layer2_block3_conv3_shift,
    layer2_block4_conv1_w, layer2_block4_conv1_shift,
    layer2_block4_conv2_w, layer2_block4_conv2_shift,
    layer2_block4_conv3_w, layer2_block4_conv3_shift,
    layer2_block5_conv1_w, layer2_block5_conv1_shift,
    layer2_block5_conv2_w, layer2_block5_conv2_shift,
    layer2_block5_conv3_w, layer2_block5_conv3_shift,
    layer3_block0_conv1_w, layer3_block0_conv1_shift,
    layer3_block0_conv2_w, layer3_block0_conv2_shift,
    layer3_block0_conv3_w, layer3_block0_conv3_shift,
    layer3_block0_downsample_w, layer3_block0_downsample_shift,
    layer3_block1_conv1_w, layer3_block1_conv1_shift,
    layer3_block1_conv2_w, layer3_block1_conv2_shift,
    layer3_block1_conv3_w, layer3_block1_conv3_shift,
    layer3_block2_conv1_w, layer3_block2_conv1_shift,
    layer3_block2_conv2_w, layer3_block2_conv2_shift,
    layer3_block2_conv3_w, layer3_block2_conv3_shift,
    fc_w, fc_b,
    classify_fc1_w, classify_fc1_b,
    classify_fc2_w, classify_fc2_b,
    classify_fc3_w, classify_fc3_b,
    regression_fc1_w, regression_fc1_b,
    regression_fc2_w, regression_fc2_b,
    regression_fc3_w, regression_fc3_b,
):
    A = dict(locals())
    return _forward(A)


# speedup vs baseline: 1.1241x; 1.1241x over previous
import functools

import jax
import jax.numpy as jnp
from jax.experimental import pallas as pl
from jax.experimental.pallas import tpu as pltpu


def _rup(x, m):
    return ((x + m - 1) // m) * m


_VMEM = 64 * 1024 * 1024


def _mm1_body(x_ref, w_ref, b_ref, *rest, relu, has_res):
    if has_res:
        r_ref, o_ref = rest
    else:
        (o_ref,) = rest
    y = jnp.dot(x_ref[...], w_ref[...], preferred_element_type=jnp.float32)
    y = y + b_ref[...]
    if has_res:
        y = y + r_ref[...].astype(jnp.float32)
    if relu:
        y = jnp.maximum(y, 0.0)
    o_ref[...] = y.astype(o_ref.dtype)


def _mmk_body(x_ref, w_ref, b_ref, *rest, nk, relu, has_res):
    if has_res:
        r_ref, o_ref, acc_ref = rest
    else:
        o_ref, acc_ref = rest
    kk = pl.program_id(2)

    @pl.when(kk == 0)
    def _():
        acc_ref[...] = jnp.zeros_like(acc_ref)

    acc_ref[...] += jnp.dot(x_ref[...], w_ref[...],
                            preferred_element_type=jnp.float32)

    @pl.when(kk == nk - 1)
    def _():
        y = acc_ref[...] + b_ref[...]
        if has_res:
            y = y + r_ref[...].astype(jnp.float32)
        if relu:
            y = jnp.maximum(y, 0.0)
        o_ref[...] = y.astype(o_ref.dtype)


def _pick_tm(m):
    for t in (512, 256, 128, 112, 64, 56, 32, 16, 8):
        if m % t == 0:
            return t
    return m


def _pick_tk(k):
    if k <= 1024:
        return k
    for t in (1024, 768, 512, 384, 256, 128):
        if k % t == 0:
            return t
    return k


@functools.lru_cache(maxsize=None)
def _mm_build(m, k, n, relu, has_res):
    tm, tk = _pick_tm(m), _pick_tk(k)
    tn = n if n <= 512 else (512 if n % 512 == 0 else n)
    nk = k // tk
    flops = 2 * m * k * n
    ba = (m * k + k * n) * 2 + m * n * 2 + 4 * n + (m * n * 2 if has_res else 0)
    if nk == 1:
        in_specs = [
            pl.BlockSpec((tm, k), lambda i, j: (i, 0)),
            pl.BlockSpec((k, tn), lambda i, j: (0, j)),
            pl.BlockSpec((1, tn), lambda i, j: (0, j)),
        ]
        if has_res:
            in_specs.append(pl.BlockSpec((tm, tn), lambda i, j: (i, j)))
        return pl.pallas_call(
            functools.partial(_mm1_body, relu=relu, has_res=has_res),
            out_shape=jax.ShapeDtypeStruct((m, n), jnp.bfloat16),
            grid=(m // tm, n // tn),
            in_specs=in_specs,
            out_specs=pl.BlockSpec((tm, tn), lambda i, j: (i, j)),
            compiler_params=pltpu.CompilerParams(
                dimension_semantics=("parallel", "parallel"),
                vmem_limit_bytes=_VMEM),
            cost_estimate=pl.CostEstimate(flops=flops, transcendentals=0,
                                          bytes_accessed=ba),
        )
    in_specs = [
        pl.BlockSpec((tm, tk), lambda i, j, kk: (i, kk)),
        pl.BlockSpec((tk, tn), lambda i, j, kk: (kk, j)),
        pl.BlockSpec((1, tn), lambda i, j, kk: (0, j)),
    ]
    if has_res:
        in_specs.append(pl.BlockSpec((tm, tn), lambda i, j, kk: (i, j)))
    return pl.pallas_call(
        functools.partial(_mmk_body, nk=nk, relu=relu, has_res=has_res),
        out_shape=jax.ShapeDtypeStruct((m, n), jnp.bfloat16),
        grid_spec=pltpu.PrefetchScalarGridSpec(
            num_scalar_prefetch=0,
            grid=(m // tm, n // tn, nk),
            in_specs=in_specs,
            out_specs=pl.BlockSpec((tm, tn), lambda i, j, kk: (i, j)),
            scratch_shapes=[pltpu.VMEM((tm, tn), jnp.float32)],
        ),
        compiler_params=pltpu.CompilerParams(
            dimension_semantics=("parallel", "parallel", "arbitrary"),
            vmem_limit_bytes=_VMEM),
        cost_estimate=pl.CostEstimate(flops=flops, transcendentals=0,
                                      bytes_accessed=ba),
    )


def _mm(x, w, bias, *, relu, res=None):
    m, k = x.shape
    n = w.shape[1]
    args = [x.astype(jnp.bfloat16), w, bias]
    if res is not None:
        args.append(res.astype(jnp.bfloat16))
    return _mm_build(m, k, n, bool(relu), res is not None)(*args)


def _c3_body(x_ref, w_ref, s_ref, o_ref, xcat_ref, *, bsz, h, w, c, hp, wpp):
    for dj in range(3):
        xcat_ref[:, 0:w, dj * c:(dj + 1) * c] = x_ref[0, :, dj:dj + w, :]
    rows = bsz * hp
    lout = rows - 2
    acc = None
    for di in range(3):
        a = xcat_ref[di:di + lout].reshape(lout * wpp, 3 * c)
        z = jnp.dot(a, w_ref[di], preferred_element_type=jnp.float32)
        acc = z if acc is None else acc + z
    y = acc + s_ref[...]
    y = jnp.maximum(y, 0.0).astype(o_ref.dtype)
    y = y.reshape(lout, wpp, o_ref.shape[-1])
    for b in range(bsz):
        o_ref[0, b * h:(b + 1) * h] = y[b * hp: b * hp + h, 0:w]


@functools.lru_cache(maxsize=None)
def _c3_build(g, bsz, h, w, c, cout, hp, wpp):
    rows = bsz * hp
    lout = rows - 2
    flops = 2 * g * lout * wpp * 3 * c * cout * 3
    ba = (g * rows * wpp * c + 9 * c * cout + g * bsz * h * w * cout) * 2
    return pl.pallas_call(
        functools.partial(_c3_body, bsz=bsz, h=h, w=w, c=c, hp=hp, wpp=wpp),
        out_shape=jax.ShapeDtypeStruct((g, bsz * h, w, cout), jnp.bfloat16),
        grid_spec=pltpu.PrefetchScalarGridSpec(
            num_scalar_prefetch=0,
            grid=(g,),
            in_specs=[
                pl.BlockSpec((1, rows, wpp, c), lambda i: (i, 0, 0, 0)),
                pl.BlockSpec((3, 3 * c, cout), lambda i: (0, 0, 0)),
                pl.BlockSpec((1, cout), lambda i: (0, 0)),
            ],
            out_specs=pl.BlockSpec((1, bsz * h, w, cout), lambda i: (i, 0, 0, 0)),
            scratch_shapes=[pltpu.VMEM((rows, wpp, 3 * c), jnp.bfloat16)],
        ),
        compiler_params=pltpu.CompilerParams(
            dimension_semantics=("parallel",),
            vmem_limit_bytes=_VMEM),
        cost_estimate=pl.CostEstimate(flops=flops, transcendentals=0,
                                      bytes_accessed=ba),
    )


def _conv3_s1(x, wfull, shift):
    n, h, w, c = x.shape
    cout = wfull.shape[1]
    w3 = wfull[:9 * c].reshape(3, 3 * c, cout)
    g = {64: 8, 128: 8, 256: 4, 512: 2}.get(c, 8)
    bsz = n // g
    hp = h + 2
    wpp = _rup(w + 2, 16)
    xp = jnp.pad(x, ((0, 0), (1, 1), (1, wpp - w - 1), (0, 0)))
    xp = xp.reshape(g, bsz * hp, wpp, c)
    out = _c3_build(g, bsz, h, w, c, cout, hp, wpp)(xp, w3, shift)
    return out.reshape(n, h, w, cout)


def _b2_body(x_ref, w1_ref, s1_ref, w2_ref, s2_ref, o_ref, t1_ref, xcat_ref,
             *, bsz, h, w, cin, c, hp, wpp):
    a = x_ref[0].reshape(bsz * h * w, cin)
    z = jnp.dot(a, w1_ref[...], preferred_element_type=jnp.float32)
    z = jnp.maximum(z + s1_ref[...], 0.0).astype(jnp.bfloat16)
    zv = z.reshape(bsz * h, w, c)
    t1_ref[...] = jnp.zeros_like(t1_ref)
    for b in range(bsz):
        t1_ref[b * hp + 1: b * hp + 1 + h, 1:1 + w, :] = zv[b * h:(b + 1) * h]
    for dj in range(3):
        xcat_ref[:, 0:w, dj * c:(dj + 1) * c] = t1_ref[:, dj:dj + w, :]
    rows = bsz * hp
    lout = rows - 2
    acc = None
    for di in range(3):
        aa = xcat_ref[di:di + lout].reshape(lout * wpp, 3 * c)
        zz = jnp.dot(aa, w2_ref[di], preferred_element_type=jnp.float32)
        acc = zz if acc is None else acc + zz
    y = jnp.maximum(acc + s2_ref[...], 0.0).astype(o_ref.dtype)
    y = y.reshape(lout, wpp, o_ref.shape[-1])
    for b in range(bsz):
        o_ref[0, b * h:(b + 1) * h] = y[b * hp: b * hp + h, 0:w]


@functools.lru_cache(maxsize=None)
def _b2_build(g, bsz, h, w, cin, c, hp, wpp):
    rows = bsz * hp
    flops = (2 * g * bsz * h * w * cin * c
             + 2 * g * (rows - 2) * wpp * 3 * c * c * 3)
    ba = (g * bsz * h * w * cin + cin * c + 9 * c * c
          + g * bsz * h * w * c) * 2
    return pl.pallas_call(
        functools.partial(_b2_body, bsz=bsz, h=h, w=w, cin=cin, c=c,
                          hp=hp, wpp=wpp),
        out_shape=jax.ShapeDtypeStruct((g, bsz * h, w, c), jnp.bfloat16),
        grid_spec=pltpu.PrefetchScalarGridSpec(
            num_scalar_prefetch=0,
            grid=(g,),
            in_specs=[
                pl.BlockSpec((1, bsz * h, w, cin), lambda i: (i, 0, 0, 0)),
                pl.BlockSpec((cin, c), lambda i: (0, 0)),
                pl.BlockSpec((1, c), lambda i: (0, 0)),
                pl.BlockSpec((3, 3 * c, c), lambda i: (0, 0, 0)),
                pl.BlockSpec((1, c), lambda i: (0, 0)),
            ],
            out_specs=pl.BlockSpec((1, bsz * h, w, c), lambda i: (i, 0, 0, 0)),
            scratch_shapes=[pltpu.VMEM((rows, wpp, c), jnp.bfloat16),
                            pltpu.VMEM((rows, wpp, 3 * c), jnp.bfloat16)],
        ),
        compiler_params=pltpu.CompilerParams(
            dimension_semantics=("parallel",),
            vmem_limit_bytes=_VMEM),
        cost_estimate=pl.CostEstimate(flops=flops, transcendentals=0,
                                      bytes_accessed=ba),
    )


def _bottleneck_front(x, w1full, s1, w2full, s2):
    n, h, w, cin = x.shape
    c = w1full.shape[1]
    w2 = w2full[:9 * c].reshape(3, 3 * c, c)
    g = {64: 8, 128: 8, 256: 4, 512: 2}.get(c, 8)
    bsz = n // g
    hp = h + 2
    wpp = _rup(w + 2, 16)
    xv = x.reshape(g, bsz * h, w, cin)
    out = _b2_build(g, bsz, h, w, cin, c, hp, wpp)(xv, w1full, s1, w2, s2)
    return out.reshape(n, h, w, c)


def _c1_body(x_ref, w_ref, s_ref, o_ref, ex_ref, ox_ref, *, ho, wo):
    v = x_ref[0]
    vr = v.reshape(115, 2, 116, 6)
    ev, od = vr[:, 0], vr[:, 1]
    for s in range(4):
        ex_ref[:, :, s * 6:(s + 1) * 6] = ev[:, s:s + 112, :]
        ox_ref[:, :, s * 6:(s + 1) * 6] = od[:, s:s + 112, :]
    acc = None
    for di in range(7):
        src = ex_ref if di % 2 == 0 else ox_ref
        off = di // 2
        a = src[off:off + 112].reshape(112 * 112, 24)
        z = jnp.dot(a, w_ref[di], preferred_element_type=jnp.float32)
        acc = z if acc is None else acc + z
    y = jnp.maximum(acc + s_ref[...], 0.0).astype(o_ref.dtype)
    y = y.reshape(112, 112, 64)
    yr = y.reshape(112, wo, 2, 64)
    a0, a1 = yr[:, :, 0], yr[:, :, 1]
    ninf = jnp.full((112, 1, 64), -jnp.inf, y.dtype)
    cw = jnp.maximum(jnp.maximum(a0, a1),
                     jnp.concatenate([ninf, a1[:, :-1]], axis=1))
    cr = cw.reshape(ho, 2, wo, 64)
    c0, c1 = cr[:, 0], cr[:, 1]
    ninf2 = jnp.full((1, wo, 64), -jnp.inf, y.dtype)
    o_ref[0] = jnp.maximum(jnp.maximum(c0, c1),
                           jnp.concatenate([ninf2, c1[:-1]], axis=0))


@functools.lru_cache(maxsize=None)
def _c1_build(n):
    return pl.pallas_call(
        functools.partial(_c1_body, ho=56, wo=56),
        out_shape=jax.ShapeDtypeStruct((n, 56, 56, 64), jnp.bfloat16),
        grid_spec=pltpu.PrefetchScalarGridSpec(
            num_scalar_prefetch=0,
            grid=(n,),
            in_specs=[
                pl.BlockSpec((1, 230, 116, 6), lambda i: (i, 0, 0, 0)),
                pl.BlockSpec((7, 24, 64), lambda i: (0, 0, 0)),
                pl.BlockSpec((1, 64), lambda i: (0, 0)),
            ],
            out_specs=pl.BlockSpec((1, 56, 56, 64), lambda i: (i, 0, 0, 0)),
            scratch_shapes=[pltpu.VMEM((115, 112, 24), jnp.bfloat16),
                            pltpu.VMEM((115, 112, 24), jnp.bfloat16)],
        ),
        compiler_params=pltpu.CompilerParams(
            dimension_semantics=("parallel",),
            vmem_limit_bytes=_VMEM),
        cost_estimate=pl.CostEstimate(
            flops=2 * n * 112 * 112 * 24 * 64 * 7, transcendentals=0,
            bytes_accessed=n * (230 * 116 * 6 + 56 * 56 * 64) * 2),
    )


def _conv1_pool(x_nchw, wfull, shift):
    n = x_nchw.shape[0]
    t = jnp.transpose(x_nchw, (0, 2, 3, 1)).astype(jnp.bfloat16)
    tp = jnp.pad(t, ((0, 0), (3, 3), (3, 5), (0, 0)))
    xq = tp.reshape(n, 230, 116, 6)
    w7 = jnp.zeros((7, 24, 64), jnp.bfloat16)
    for di in range(7):
        for s in range(4):
            for e in range(2):
                dj = 2 * s + e
                if dj < 7:
                    blk = wfull[di * 21 + dj * 3: di * 21 + dj * 3 + 3, :64]
                    w7 = w7.at[di, s * 6 + e * 3: s * 6 + e * 3 + 3, :].set(blk)
    return _c1_build(n)(xq, w7, shift)


def _c3s2_body(x_ref, w_ref, s_ref, o_ref, ecat_ref, ocat_ref,
               *, bsz, ho, wo, c, pairs, wpr, woc):
    rows = bsz * 2 * pairs
    v = x_ref[0].reshape(bsz * pairs, 2, wpr, 2 * c)
    ev, od = v[:, 0], v[:, 1]
    ecat_ref[:, 0:wpr, 0:2 * c] = ev
    ocat_ref[:, 0:wpr, 0:2 * c] = od
    ecat_ref[:, 0:wpr - 1, 2 * c:3 * c] = ev[:, 1:wpr, 0:c]
    ocat_ref[:, 0:wpr - 1, 2 * c:3 * c] = od[:, 1:wpr, 0:c]
    lout = bsz * pairs - 1
    acc = None
    for di in range(3):
        src = ecat_ref if di % 2 == 0 else ocat_ref
        a = src[di // 2:di // 2 + lout].reshape(lout * woc, 3 * c)
        z = jnp.dot(a, w_ref[di], preferred_element_type=jnp.float32)
        acc = z if acc is None else acc + z
    y = jnp.maximum(acc + s_ref[...], 0.0).astype(o_ref.dtype)
    y = y.reshape(lout, woc, o_ref.shape[-1])
    for b in range(bsz):
        o_ref[0, b * ho:(b + 1) * ho] = y[b * pairs: b * pairs + ho, 0:wo]


@functools.lru_cache(maxsize=None)
def _c3s2_build(g, bsz, ho, wo, c, cout, pairs, wpr, woc):
    rows = bsz * 2 * pairs
    lout = bsz * pairs - 1
    return pl.pallas_call(
        functools.partial(_c3s2_body, bsz=bsz, ho=ho, wo=wo, c=c,
                          pairs=pairs, wpr=wpr, woc=woc),
        out_shape=jax.ShapeDtypeStruct((g, bsz * ho, wo, cout), jnp.bfloat16),
        grid_spec=pltpu.PrefetchScalarGridSpec(
            num_scalar_prefetch=0,
            grid=(g,),
            in_specs=[
                pl.BlockSpec((1, rows, wpr, 2 * c), lambda i: (i, 0, 0, 0)),
                pl.BlockSpec((3, 3 * c, cout), lambda i: (0, 0, 0)),
                pl.BlockSpec((1, cout), lambda i: (0, 0)),
            ],
            out_specs=pl.BlockSpec((1, bsz * ho, wo, cout), lambda i: (i, 0, 0, 0)),
            scratch_shapes=[pltpu.VMEM((bsz * pairs, woc, 3 * c), jnp.bfloat16),
                            pltpu.VMEM((bsz * pairs, woc, 3 * c), jnp.bfloat16)],
        ),
        compiler_params=pltpu.CompilerParams(
            dimension_semantics=("parallel",),
            vmem_limit_bytes=_VMEM),
        cost_estimate=pl.CostEstimate(
            flops=2 * g * lout * woc * 3 * c * cout * 3, transcendentals=0,
            bytes_accessed=(g * rows * wpr * 2 * c + 9 * c * cout
                            + g * bsz * ho * wo * cout) * 2),
    )


def _conv3_s2(x, wfull, shift):
    n, h, w, c = x.shape
    cout = wfull.shape[1]
    ho, wo = h // 2, w // 2
    pairs = (h + 2) // 2
    wpr = (w + 2 + 1) // 2
    woc = _rup(wo, 16)
    w3 = wfull[:9 * c].reshape(3, 3 * c, cout)
    g = {128: 8, 256: 4, 512: 2}.get(c, 4)
    bsz = n // g
    xp = jnp.pad(x, ((0, 0), (1, 1), (1, 2 * wpr - w - 1), (0, 0)))
    xp = xp.reshape(g, bsz * 2 * pairs, wpr, 2 * c)
    out = _c3s2_build(g, bsz, ho, wo, c, cout, pairs, wpr, woc)(xp, w3, shift)
    return out.reshape(n, ho, wo, cout)


def _ds_body(x_ref, w_ref, s_ref, o_ref, *, ho, wo, c):
    v = x_ref[0].reshape(ho, 2, wo, 2 * c)[:, 0]
    a = v.reshape(ho * wo, 2 * c)
    z = jnp.dot(a, w_ref[...], preferred_element_type=jnp.float32)
    z = z + s_ref[...]
    o_ref[0] = z.astype(o_ref.dtype).reshape(ho, wo, o_ref.shape[-1])


@functools.lru_cache(maxsize=None)
def _ds_build(n, h, wo, c, cout):
    return pl.pallas_call(
        functools.partial(_ds_body, ho=h // 2, wo=wo, c=c),
        out_shape=jax.ShapeDtypeStruct((n, h // 2, wo, cout), jnp.bfloat16),
        grid_spec=pltpu.PrefetchScalarGridSpec(
            num_scalar_prefetch=0,
            grid=(n,),
            in_specs=[
                pl.BlockSpec((1, h, wo, 2 * c), lambda i: (i, 0, 0, 0)),
                pl.BlockSpec((2 * c, cout), lambda i: (0, 0)),
                pl.BlockSpec((1, cout), lambda i: (0, 0)),
            ],
            out_specs=pl.BlockSpec((1, h // 2, wo, cout), lambda i: (i, 0, 0, 0)),
        ),
        compiler_params=pltpu.CompilerParams(
            dimension_semantics=("parallel",),
            vmem_limit_bytes=_VMEM),
        cost_estimate=pl.CostEstimate(
            flops=2 * n * (h // 2) * wo * 2 * c * cout, transcendentals=0,
            bytes_accessed=(n * h * wo * c * 2 + 2 * c * cout
                            + n * (h // 2) * wo * cout) * 2),
    )


def _downsample(x, wfull, shift):
    n, h, w, c = x.shape
    cout = wfull.shape[1]
    wz = jnp.concatenate([wfull, jnp.zeros_like(wfull)], axis=0)
    xv = x.reshape(n, h, w // 2, 2 * c)
    return _ds_build(n, h, w // 2, c, cout)(xv, wz, shift)


def _pool_body(ee_ref, eo_ref, oe_ref, oo_ref, o_ref, *, ho, wo):
    ee, eo, oe, oo = ee_ref[0], eo_ref[0], oe_ref[0], oo_ref[0]
    a = jnp.maximum(jnp.maximum(ee[:, :wo], eo[:, :wo]), ee[:, 1:wo + 1])
    b = jnp.maximum(jnp.maximum(oe[:, :wo], oo[:, :wo]), oe[:, 1:wo + 1])
    o_ref[0] = jnp.maximum(jnp.maximum(a[:ho], b[:ho]), a[1:ho + 1])


@functools.lru_cache(maxsize=None)
def _pool_build(n, he, we, ho, wo, c):
    spec = pl.BlockSpec((1, he, we, c), lambda i: (i, 0, 0, 0))
    return pl.pallas_call(
        functools.partial(_pool_body, ho=ho, wo=wo),
        out_shape=jax.ShapeDtypeStruct((n, ho, wo, c), jnp.bfloat16),
        grid=(n,),
        in_specs=[spec, spec, spec, spec],
        out_specs=pl.BlockSpec((1, ho, wo, c), lambda i: (i, 0, 0, 0)),
        compiler_params=pltpu.CompilerParams(
            dimension_semantics=("parallel",),
            vmem_limit_bytes=_VMEM),
    )


def _maxpool(x):
    n, h, w, c = x.shape
    ho, wo = (h - 1) // 2 + 1, (w - 1) // 2 + 1
    he, we = ho + 1, wo + 1
    xp = jnp.pad(x, ((0, 0), (1, 2 * he - h - 1), (1, 2 * we - w - 1), (0, 0)),
                 constant_values=-jnp.inf)
    ee = xp[:, 0::2, 0::2]
    eo = xp[:, 0::2, 1::2]
    oe = xp[:, 1::2, 0::2]
    oo = xp[:, 1::2, 1::2]
    return _pool_build(n, he, we, ho, wo, c)(ee, eo, oe, oo)


def _tail_body(x_ref, fw_ref, fb_ref, w1_ref, b1_ref, w2_ref, b2_ref,
               w3c_ref, b3c_ref, w3r_ref, b3r_ref, c_ref, r_ref, *, inv_hw, hm):
    x = x_ref[...].astype(jnp.float32)
    pooled = jnp.sum(x, axis=1) * inv_hw
    feats = jnp.dot(pooled.astype(jnp.bfloat16), fw_ref[...],
                    preferred_element_type=jnp.float32) + fb_ref[...]
    h1 = jnp.dot(feats.astype(jnp.bfloat16), w1_ref[...],
                 preferred_element_type=jnp.float32) + b1_ref[...]
    h1 = jnp.maximum(h1, 0.0)
    h2 = jnp.dot(h1.astype(jnp.bfloat16), w2_ref[...],
                 preferred_element_type=jnp.float32) + b2_ref[...]
    h2 = jnp.maximum(h2, 0.0).astype(jnp.bfloat16)
    c_ref[...] = jnp.dot(h2[:, :hm], w3c_ref[...],
                         preferred_element_type=jnp.float32) + b3c_ref[...]
    r_ref[...] = jnp.dot(h2[:, hm:], w3r_ref[...],
                         preferred_element_type=jnp.float32) + b3r_ref[...]


@functools.lru_cache(maxsize=None)
def _tail_build(batch, hw, nc, nr):
    vmem = lambda: pl.BlockSpec(memory_space=pltpu.MemorySpace.VMEM)
    return pl.pallas_call(
        functools.partial(_tail_body, inv_hw=1.0 / hw, hm=32),
        out_shape=(jax.ShapeDtypeStruct((batch, nc), jnp.float32),
                   jax.ShapeDtypeStruct((batch, nr), jnp.float32)),
        in_specs=[vmem() for _ in range(11)],
        out_specs=(vmem(), vmem()),
        compiler_params=pltpu.CompilerParams(vmem_limit_bytes=_VMEM),
    )


def _tail(x, A):
    n, h, w, c = x.shape
    xr = x.reshape(n, h * w, c)
    cw1, cb1 = A["classify_fc1_w"], A["classify_fc1_b"]
    rw1, rb1 = A["regression_fc1_w"], A["regression_fc1_b"]
    cw2, cb2 = A["classify_fc2_w"], A["classify_fc2_b"]
    rw2, rb2 = A["regression_fc2_w"], A["regression_fc2_b"]
    d1 = cw1.shape[1]
    w1 = jnp.concatenate([cw1, rw1], axis=1)
    b1 = jnp.concatenate([cb1, rb1], axis=1)
    z = jnp.zeros_like(cw2)
    w2 = jnp.concatenate(
        [jnp.concatenate([cw2, z], axis=1), jnp.concatenate([z, rw2], axis=1)],
        axis=0)
    b2 = jnp.concatenate([cb2, rb2], axis=1)
    nc = A["classify_fc3_w"].shape[1]
    nr = A["regression_fc3_w"].shape[1]
    return _tail_build(n, h * w, nc, nr)(
        xr, A["fc_w"], A["fc_b"], w1, b1, w2, b2,
        A["classify_fc3_w"], A["classify_fc3_b"],
        A["regression_fc3_w"], A["regression_fc3_b"])


_ARCH = [(64, 3, 1), (128, 4, 2), (256, 6, 2), (512, 3, 2)]


def _forward(A):
    n = A["x"].shape[0]
    x = _conv1_pool(A["x"], A["conv1_w"], A["conv1_shift"])
    h = w = x.shape[1]
    cin = x.shape[3]

    for li, (planes, nblocks, lstride) in enumerate(_ARCH):
        for bi in range(nblocks):
            s = lstride if bi == 0 else 1
            pre = "layer%d_block%d_" % (li, bi)
            hn, wn = h // s, w // s
            if bi == 0:
                if s == 2:
                    idm = _downsample(x, A[pre + "downsample_w"],
                                      A[pre + "downsample_shift"])
                    idm = idm.reshape(n * hn * wn, -1)
                else:
                    idm = _mm(x.reshape(n * h * w, cin),
                              A[pre + "downsample_w"],
                              A[pre + "downsample_shift"], relu=False)
            else:
                idm = x.reshape(n * h * w, cin)
            if s == 1:
                y = _bottleneck_front(x, A[pre + "conv1_w"],
                                      A[pre + "conv1_shift"],
                                      A[pre + "conv2_w"],
                                      A[pre + "conv2_shift"])
            else:
                y = _mm(x.reshape(n * h * w, cin),
                        A[pre + "conv1_w"], A[pre + "conv1_shift"], relu=True)
                y = _conv3_s2(y.reshape(n, h, w, planes),
                              A[pre + "conv2_w"], A[pre + "conv2_shift"])
            y = _mm(y.reshape(n * hn * wn, planes),
                    A[pre + "conv3_w"], A[pre + "conv3_shift"],
                    relu=True, res=idm)
            cin = 4 * planes
            h, w = hn, wn
            x = y.reshape(n, h, w, cin)

    return _tail(x, A)


def kernel(
    x,
    conv1_w, conv1_shift,
    layer0_block0_conv1_w, layer0_block0_conv1_shift,
    layer0_block0_conv2_w, layer0_block0_conv2_shift,
    layer0_block0_conv3_w, layer0_block0_conv3_shift,
    layer0_block0_downsample_w, layer0_block0_downsample_shift,
    layer0_block1_conv1_w, layer0_block1_conv1_shift,
    layer0_block1_conv2_w, layer0_block1_conv2_shift,
    layer0_block1_conv3_w, layer0_block1_conv3_shift,
    layer0_block2_conv1_w, layer0_block2_conv1_shift,
    layer0_block2_conv2_w, layer0_block2_conv2_shift,
    layer0_block2_conv3_w, layer0_block2_conv3_shift,
    layer1_block0_conv1_w, layer1_block0_conv1_shift,
    layer1_block0_conv2_w, layer1_block0_conv2_shift,
    layer1_block0_conv3_w, layer1_block0_conv3_shift,
    layer1_block0_downsample_w, layer1_block0_downsample_shift,
    layer1_block1_conv1_w, layer1_block1_conv1_shift,
    layer1_block1_conv2_w, layer1_block1_conv2_shift,
    layer1_block1_conv3_w, layer1_block1_conv3_shift,
    layer1_block2_conv1_w, layer1_block2_conv1_shift,
    layer1_block2_conv2_w, layer1_block2_conv2_shift,
    layer1_block2_conv3_w, layer1_block2_conv3_shift,
    layer1_block3_conv1_w, layer1_block3_conv1_shift,
    layer1_block3_conv2_w, layer1_block3_conv2_shift,
    layer1_block3_conv3_w, layer1_block3_conv3_shift,
    layer2_block0_conv1_w, layer2_block0_conv1_shift,
    layer2_block0_conv2_w, layer2_block0_conv2_shift,
    layer2_block0_conv3_w, layer2_block0_conv3_shift,
    layer2_block0_downsample_w, layer2_block0_downsample_shift,
    layer2_block1_conv1_w, layer2_block1_conv1_shift,
    layer2_block1_conv2_w, layer2_block1_conv2_shift,
    layer2_block1_conv3_w, layer2_block1_conv3_shift,
    layer2_block2_conv1_w, layer2_block2_conv1_shift,
    layer2_block2_conv2_w, layer2_block2_conv2_shift,
    layer2_block2_conv3_w, layer2_block2_conv3_shift,
    layer2_block3_conv1_w, layer2_block3_conv1_shift,
    layer2_block3_conv2_w, layer2_block3_conv2_shift,
    layer2_block3_conv3_w, layer2_block3_conv3_shift,
    layer2_block4_conv1_w, layer2_block4_conv1_shift,
    layer2_block4_conv2_w, layer2_block4_conv2_shift,
    layer2_block4_conv3_w, layer2_block4_conv3_shift,
    layer2_block5_conv1_w, layer2_block5_conv1_shift,
    layer2_block5_conv2_w, layer2_block5_conv2_shift,
    layer2_block5_conv3_w, layer2_block5_conv3_shift,
    layer3_block0_conv1_w, layer3_block0_conv1_shift,
    layer3_block0_conv2_w, layer3_block0_conv2_shift,
    layer3_block0_conv3_w, layer3_block0_conv3_shift,
    layer3_block0_downsample_w, layer3_block0_downsample_shift,
    layer3_block1_conv1_w, layer3_block1_conv1_shift,
    layer3_block1_conv2_w, layer3_block1_conv2_shift,
    layer3_block1_conv3_w, layer3_block1_conv3_shift,
    layer3_block2_conv1_w, layer3_block2_conv1_shift,
    layer3_block2_conv2_w, layer3_block2_conv2_shift,
    layer3_block2_conv3_w, layer3_block2_conv3_shift,
    fc_w, fc_b,
    classify_fc1_w, classify_fc1_b,
    classify_fc2_w, classify_fc2_b,
    classify_fc3_w, classify_fc3_b,
    regression_fc1_w, regression_fc1_b,
    regression_fc2_w, regression_fc2_b,
    regression_fc3_w, regression_fc3_b,
):
    A = dict(locals())
    return _forward(A)


# speedup vs baseline: 1.4140x; 1.2579x over previous
import functools

import jax
import jax.numpy as jnp
from jax.experimental import pallas as pl
from jax.experimental.pallas import tpu as pltpu


def _rup(x, m):
    return ((x + m - 1) // m) * m


_VMEM = 64 * 1024 * 1024


def _mm1_body(x_ref, w_ref, b_ref, *rest, relu, has_res):
    if has_res:
        r_ref, o_ref = rest
    else:
        (o_ref,) = rest
    y = jnp.dot(x_ref[...], w_ref[...], preferred_element_type=jnp.float32)
    y = y + b_ref[...]
    if has_res:
        y = y + r_ref[...].astype(jnp.float32)
    if relu:
        y = jnp.maximum(y, 0.0)
    o_ref[...] = y.astype(o_ref.dtype)


def _mmk_body(x_ref, w_ref, b_ref, *rest, nk, relu, has_res):
    if has_res:
        r_ref, o_ref, acc_ref = rest
    else:
        o_ref, acc_ref = rest
    kk = pl.program_id(2)

    @pl.when(kk == 0)
    def _():
        acc_ref[...] = jnp.zeros_like(acc_ref)

    acc_ref[...] += jnp.dot(x_ref[...], w_ref[...],
                            preferred_element_type=jnp.float32)

    @pl.when(kk == nk - 1)
    def _():
        y = acc_ref[...] + b_ref[...]
        if has_res:
            y = y + r_ref[...].astype(jnp.float32)
        if relu:
            y = jnp.maximum(y, 0.0)
        o_ref[...] = y.astype(o_ref.dtype)


def _pick_tm(m):
    for t in (512, 256, 128, 112, 64, 56, 32, 16, 8):
        if m % t == 0:
            return t
    return m


def _pick_tk(k):
    if k <= 1024:
        return k
    for t in (1024, 768, 512, 384, 256, 128):
        if k % t == 0:
            return t
    return k


@functools.lru_cache(maxsize=None)
def _mm_build(m, k, n, relu, has_res):
    tm, tk = _pick_tm(m), _pick_tk(k)
    tn = n if n <= 512 else (512 if n % 512 == 0 else n)
    nk = k // tk
    flops = 2 * m * k * n
    ba = (m * k + k * n) * 2 + m * n * 2 + 4 * n + (m * n * 2 if has_res else 0)
    if nk == 1:
        in_specs = [
            pl.BlockSpec((tm, k), lambda i, j: (i, 0)),
            pl.BlockSpec((k, tn), lambda i, j: (0, j)),
            pl.BlockSpec((1, tn), lambda i, j: (0, j)),
        ]
        if has_res:
            in_specs.append(pl.BlockSpec((tm, tn), lambda i, j: (i, j)))
        return pl.pallas_call(
            functools.partial(_mm1_body, relu=relu, has_res=has_res),
            out_shape=jax.ShapeDtypeStruct((m, n), jnp.bfloat16),
            grid=(m // tm, n // tn),
            in_specs=in_specs,
            out_specs=pl.BlockSpec((tm, tn), lambda i, j: (i, j)),
            compiler_params=pltpu.CompilerParams(
                dimension_semantics=("parallel", "parallel"),
                vmem_limit_bytes=_VMEM),
            cost_estimate=pl.CostEstimate(flops=flops, transcendentals=0,
                                          bytes_accessed=ba),
        )
    in_specs = [
        pl.BlockSpec((tm, tk), lambda i, j, kk: (i, kk)),
        pl.BlockSpec((tk, tn), lambda i, j, kk: (kk, j)),
        pl.BlockSpec((1, tn), lambda i, j, kk: (0, j)),
    ]
    if has_res:
        in_specs.append(pl.BlockSpec((tm, tn), lambda i, j, kk: (i, j)))
    return pl.pallas_call(
        functools.partial(_mmk_body, nk=nk, relu=relu, has_res=has_res),
        out_shape=jax.ShapeDtypeStruct((m, n), jnp.bfloat16),
        grid_spec=pltpu.PrefetchScalarGridSpec(
            num_scalar_prefetch=0,
            grid=(m // tm, n // tn, nk),
            in_specs=in_specs,
            out_specs=pl.BlockSpec((tm, tn), lambda i, j, kk: (i, j)),
            scratch_shapes=[pltpu.VMEM((tm, tn), jnp.float32)],
        ),
        compiler_params=pltpu.CompilerParams(
            dimension_semantics=("parallel", "parallel", "arbitrary"),
            vmem_limit_bytes=_VMEM),
        cost_estimate=pl.CostEstimate(flops=flops, transcendentals=0,
                                      bytes_accessed=ba),
    )


def _mm(x, w, bias, *, relu, res=None):
    m, k = x.shape
    n = w.shape[1]
    args = [x.astype(jnp.bfloat16), w, bias]
    if res is not None:
        args.append(res.astype(jnp.bfloat16))
    return _mm_build(m, k, n, bool(relu), res is not None)(*args)


def _c3_body(x_ref, w_ref, s_ref, o_ref, xcat_ref, *, bsz, h, w, c, hp, wpp):
    for dj in range(3):
        xcat_ref[:, 0:w, dj * c:(dj + 1) * c] = x_ref[0, :, dj:dj + w, :]
    rows = bsz * hp
    lout = rows - 2
    acc = None
    for di in range(3):
        a = xcat_ref[di:di + lout].reshape(lout * wpp, 3 * c)
        z = jnp.dot(a, w_ref[di], preferred_element_type=jnp.float32)
        acc = z if acc is None else acc + z
    y = acc + s_ref[...]
    y = jnp.maximum(y, 0.0).astype(o_ref.dtype)
    y = y.reshape(lout, wpp, o_ref.shape[-1])
    for b in range(bsz):
        o_ref[0, b * h:(b + 1) * h] = y[b * hp: b * hp + h, 0:w]


@functools.lru_cache(maxsize=None)
def _c3_build(g, bsz, h, w, c, cout, hp, wpp):
    rows = bsz * hp
    lout = rows - 2
    flops = 2 * g * lout * wpp * 3 * c * cout * 3
    ba = (g * rows * wpp * c + 9 * c * cout + g * bsz * h * w * cout) * 2
    return pl.pallas_call(
        functools.partial(_c3_body, bsz=bsz, h=h, w=w, c=c, hp=hp, wpp=wpp),
        out_shape=jax.ShapeDtypeStruct((g, bsz * h, w, cout), jnp.bfloat16),
        grid_spec=pltpu.PrefetchScalarGridSpec(
            num_scalar_prefetch=0,
            grid=(g,),
            in_specs=[
                pl.BlockSpec((1, rows, wpp, c), lambda i: (i, 0, 0, 0)),
                pl.BlockSpec((3, 3 * c, cout), lambda i: (0, 0, 0)),
                pl.BlockSpec((1, cout), lambda i: (0, 0)),
            ],
            out_specs=pl.BlockSpec((1, bsz * h, w, cout), lambda i: (i, 0, 0, 0)),
            scratch_shapes=[pltpu.VMEM((rows, wpp, 3 * c), jnp.bfloat16)],
        ),
        compiler_params=pltpu.CompilerParams(
            dimension_semantics=("parallel",),
            vmem_limit_bytes=_VMEM),
        cost_estimate=pl.CostEstimate(flops=flops, transcendentals=0,
                                      bytes_accessed=ba),
    )


def _conv3_s1(x, wfull, shift):
    n, h, w, c = x.shape
    cout = wfull.shape[1]
    w3 = wfull[:9 * c].reshape(3, 3 * c, cout)
    g = {64: 8, 128: 8, 256: 4, 512: 2}.get(c, 8)
    bsz = n // g
    hp = h + 2
    wpp = _rup(w + 2, 16)
    xp = jnp.pad(x, ((0, 0), (1, 1), (1, wpp - w - 1), (0, 0)))
    xp = xp.reshape(g, bsz * hp, wpp, c)
    out = _c3_build(g, bsz, h, w, c, cout, hp, wpp)(xp, w3, shift)
    return out.reshape(n, h, w, cout)


def _b2_body(x_ref, w1_ref, s1_ref, w2_ref, s2_ref, o_ref, t1_ref, xcat_ref,
             *, bsz, h, w, cin, c, hp, wpp):
    a = x_ref[0].reshape(bsz * h * w, cin)
    z = jnp.dot(a, w1_ref[...], preferred_element_type=jnp.float32)
    z = jnp.maximum(z + s1_ref[...], 0.0).astype(jnp.bfloat16)
    zv = z.reshape(bsz * h, w, c)
    t1_ref[...] = jnp.zeros_like(t1_ref)
    for b in range(bsz):
        t1_ref[b * hp + 1: b * hp + 1 + h, 1:1 + w, :] = zv[b * h:(b + 1) * h]
    for dj in range(3):
        xcat_ref[:, 0:w, dj * c:(dj + 1) * c] = t1_ref[:, dj:dj + w, :]
    rows = bsz * hp
    lout = rows - 2
    acc = None
    for di in range(3):
        aa = xcat_ref[di:di + lout].reshape(lout * wpp, 3 * c)
        zz = jnp.dot(aa, w2_ref[di], preferred_element_type=jnp.float32)
        acc = zz if acc is None else acc + zz
    y = jnp.maximum(acc + s2_ref[...], 0.0).astype(o_ref.dtype)
    y = y.reshape(lout, wpp, o_ref.shape[-1])
    for b in range(bsz):
        o_ref[0, b * h:(b + 1) * h] = y[b * hp: b * hp + h, 0:w]


@functools.lru_cache(maxsize=None)
def _b2_build(g, bsz, h, w, cin, c, hp, wpp):
    rows = bsz * hp
    flops = (2 * g * bsz * h * w * cin * c
             + 2 * g * (rows - 2) * wpp * 3 * c * c * 3)
    ba = (g * bsz * h * w * cin + cin * c + 9 * c * c
          + g * bsz * h * w * c) * 2
    return pl.pallas_call(
        functools.partial(_b2_body, bsz=bsz, h=h, w=w, cin=cin, c=c,
                          hp=hp, wpp=wpp),
        out_shape=jax.ShapeDtypeStruct((g, bsz * h, w, c), jnp.bfloat16),
        grid_spec=pltpu.PrefetchScalarGridSpec(
            num_scalar_prefetch=0,
            grid=(g,),
            in_specs=[
                pl.BlockSpec((1, bsz * h, w, cin), lambda i: (i, 0, 0, 0)),
                pl.BlockSpec((cin, c), lambda i: (0, 0)),
                pl.BlockSpec((1, c), lambda i: (0, 0)),
                pl.BlockSpec((3, 3 * c, c), lambda i: (0, 0, 0)),
                pl.BlockSpec((1, c), lambda i: (0, 0)),
            ],
            out_specs=pl.BlockSpec((1, bsz * h, w, c), lambda i: (i, 0, 0, 0)),
            scratch_shapes=[pltpu.VMEM((rows, wpp, c), jnp.bfloat16),
                            pltpu.VMEM((rows, wpp, 3 * c), jnp.bfloat16)],
        ),
        compiler_params=pltpu.CompilerParams(
            dimension_semantics=("parallel",),
            vmem_limit_bytes=_VMEM),
        cost_estimate=pl.CostEstimate(flops=flops, transcendentals=0,
                                      bytes_accessed=ba),
    )


def _bneck_body(x_ref, w1_ref, s1_ref, w2_ref, s2_ref, w3_ref, s3_ref,
                o_ref, t1_ref, xcat_ref, t2_ref,
                *, bsz, h, w, cin, c, hp, wpp):
    a = x_ref[0].reshape(bsz * h * w, cin)
    z = jnp.dot(a, w1_ref[...], preferred_element_type=jnp.float32)
    z = jnp.maximum(z + s1_ref[...], 0.0).astype(jnp.bfloat16)
    zv = z.reshape(bsz * h, w, c)
    t1_ref[...] = jnp.zeros_like(t1_ref)
    for b in range(bsz):
        t1_ref[b * hp + 1: b * hp + 1 + h, 1:1 + w, :] = zv[b * h:(b + 1) * h]
    for dj in range(3):
        xcat_ref[:, 0:w, dj * c:(dj + 1) * c] = t1_ref[:, dj:dj + w, :]
    rows = bsz * hp
    lout = rows - 2
    acc = None
    for di in range(3):
        aa = xcat_ref[di:di + lout].reshape(lout * wpp, 3 * c)
        zz = jnp.dot(aa, w2_ref[di], preferred_element_type=jnp.float32)
        acc = zz if acc is None else acc + zz
    y = jnp.maximum(acc + s2_ref[...], 0.0).astype(jnp.bfloat16)
    y = y.reshape(lout, wpp, c)
    for b in range(bsz):
        t2_ref[b * h:(b + 1) * h] = y[b * hp: b * hp + h, 0:w]
    a3 = t2_ref[...].reshape(bsz * h * w, c)
    z3 = jnp.dot(a3, w3_ref[...], preferred_element_type=jnp.float32)
    z3 = z3 + s3_ref[...] + x_ref[0].reshape(bsz * h * w, cin).astype(jnp.float32)
    z3 = jnp.maximum(z3, 0.0).astype(o_ref.dtype)
    o_ref[0] = z3.reshape(bsz * h, w, cin)


@functools.lru_cache(maxsize=None)
def _bneck_build(g, bsz, h, w, cin, c, hp, wpp):
    rows = bsz * hp
    m = bsz * h * w
    flops = (2 * g * m * cin * c + 2 * g * (rows - 2) * wpp * 3 * c * c * 3
             + 2 * g * m * c * cin)
    ba = (2 * g * m * cin + cin * c * 2 + 9 * c * c + g * m * c) * 2
    return pl.pallas_call(
        functools.partial(_bneck_body, bsz=bsz, h=h, w=w, cin=cin, c=c,
                          hp=hp, wpp=wpp),
        out_shape=jax.ShapeDtypeStruct((g, bsz * h, w, cin), jnp.bfloat16),
        grid_spec=pltpu.PrefetchScalarGridSpec(
            num_scalar_prefetch=0,
            grid=(g,),
            in_specs=[
                pl.BlockSpec((1, bsz * h, w, cin), lambda i: (i, 0, 0, 0)),
                pl.BlockSpec((cin, c), lambda i: (0, 0)),
                pl.BlockSpec((1, c), lambda i: (0, 0)),
                pl.BlockSpec((3, 3 * c, c), lambda i: (0, 0, 0)),
                pl.BlockSpec((1, c), lambda i: (0, 0)),
                pl.BlockSpec((c, cin), lambda i: (0, 0)),
                pl.BlockSpec((1, cin), lambda i: (0, 0)),
            ],
            out_specs=pl.BlockSpec((1, bsz * h, w, cin), lambda i: (i, 0, 0, 0)),
            scratch_shapes=[pltpu.VMEM((rows, wpp, c), jnp.bfloat16),
                            pltpu.VMEM((rows, wpp, 3 * c), jnp.bfloat16),
                            pltpu.VMEM((bsz * h, w, c), jnp.bfloat16)],
        ),
        compiler_params=pltpu.CompilerParams(
            dimension_semantics=("parallel",),
            vmem_limit_bytes=_VMEM),
        cost_estimate=pl.CostEstimate(flops=flops, transcendentals=0,
                                      bytes_accessed=ba),
    )


def _bottleneck(x, w1full, s1, w2full, s2, w3full, s3):
    n, h, w, cin = x.shape
    c = w1full.shape[1]
    w2 = w2full[:9 * c].reshape(3, 3 * c, c)
    g = {64: 8, 128: 8, 256: 4, 512: 2}.get(c, 8)
    bsz = n // g
    hp = h + 2
    wpp = _rup(w + 2, 16)
    xv = x.reshape(g, bsz * h, w, cin)
    out = _bneck_build(g, bsz, h, w, cin, c, hp, wpp)(
        xv, w1full, s1, w2, s2, w3full, s3)
    return out.reshape(n, h, w, cin)


def _bottleneck_front(x, w1full, s1, w2full, s2):
    n, h, w, cin = x.shape
    c = w1full.shape[1]
    w2 = w2full[:9 * c].reshape(3, 3 * c, c)
    g = {64: 8, 128: 8, 256: 4, 512: 2}.get(c, 8)
    bsz = n // g
    hp = h + 2
    wpp = _rup(w + 2, 16)
    xv = x.reshape(g, bsz * h, w, cin)
    out = _b2_build(g, bsz, h, w, cin, c, hp, wpp)(xv, w1full, s1, w2, s2)
    return out.reshape(n, h, w, c)


def _c1_body(x_ref, w_ref, s_ref, o_ref, ex_ref, ox_ref, *, ho, wo):
    v = x_ref[0]
    vr = v.reshape(115, 2, 116, 6)
    ev, od = vr[:, 0], vr[:, 1]
    for s in range(4):
        ex_ref[:, :, s * 6:(s + 1) * 6] = ev[:, s:s + 112, :]
        ox_ref[:, :, s * 6:(s + 1) * 6] = od[:, s:s + 112, :]
    acc = None
    for di in range(7):
        src = ex_ref if di % 2 == 0 else ox_ref
        off = di // 2
        a = src[off:off + 112].reshape(112 * 112, 24)
        z = jnp.dot(a, w_ref[di], preferred_element_type=jnp.float32)
        acc = z if acc is None else acc + z
    y = jnp.maximum(acc + s_ref[...], 0.0).astype(o_ref.dtype)
    y = y.reshape(112, 112, 64)
    yr = y.reshape(112, wo, 2, 64)
    a0, a1 = yr[:, :, 0], yr[:, :, 1]
    ninf = jnp.full((112, 1, 64), -jnp.inf, y.dtype)
    cw = jnp.maximum(jnp.maximum(a0, a1),
                     jnp.concatenate([ninf, a1[:, :-1]], axis=1))
    cr = cw.reshape(ho, 2, wo, 64)
    c0, c1 = cr[:, 0], cr[:, 1]
    ninf2 = jnp.full((1, wo, 64), -jnp.inf, y.dtype)
    o_ref[0] = jnp.maximum(jnp.maximum(c0, c1),
                           jnp.concatenate([ninf2, c1[:-1]], axis=0))


@functools.lru_cache(maxsize=None)
def _c1_build(n):
    return pl.pallas_call(
        functools.partial(_c1_body, ho=56, wo=56),
        out_shape=jax.ShapeDtypeStruct((n, 56, 56, 64), jnp.bfloat16),
        grid_spec=pltpu.PrefetchScalarGridSpec(
            num_scalar_prefetch=0,
            grid=(n,),
            in_specs=[
                pl.BlockSpec((1, 230, 116, 6), lambda i: (i, 0, 0, 0)),
                pl.BlockSpec((7, 24, 64), lambda i: (0, 0, 0)),
                pl.BlockSpec((1, 64), lambda i: (0, 0)),
            ],
            out_specs=pl.BlockSpec((1, 56, 56, 64), lambda i: (i, 0, 0, 0)),
            scratch_shapes=[pltpu.VMEM((115, 112, 24), jnp.bfloat16),
                            pltpu.VMEM((115, 112, 24), jnp.bfloat16)],
        ),
        compiler_params=pltpu.CompilerParams(
            dimension_semantics=("parallel",),
            vmem_limit_bytes=_VMEM),
        cost_estimate=pl.CostEstimate(
            flops=2 * n * 112 * 112 * 24 * 64 * 7, transcendentals=0,
            bytes_accessed=n * (230 * 116 * 6 + 56 * 56 * 64) * 2),
    )


def _conv1_pool(x_nchw, wfull, shift):
    n = x_nchw.shape[0]
    t = jnp.transpose(x_nchw, (0, 2, 3, 1)).astype(jnp.bfloat16)
    tp = jnp.pad(t, ((0, 0), (3, 3), (3, 5), (0, 0)))
    xq = tp.reshape(n, 230, 116, 6)
    w7 = jnp.zeros((7, 24, 64), jnp.bfloat16)
    for di in range(7):
        for s in range(4):
            for e in range(2):
                dj = 2 * s + e
                if dj < 7:
                    blk = wfull[di * 21 + dj * 3: di * 21 + dj * 3 + 3, :64]
                    w7 = w7.at[di, s * 6 + e * 3: s * 6 + e * 3 + 3, :].set(blk)
    return _c1_build(n)(xq, w7, shift)


def _c3s2_body(x_ref, w_ref, s_ref, o_ref, ecat_ref, ocat_ref,
               *, bsz, ho, wo, c, pairs, wpr, woc):
    rows = bsz * 2 * pairs
    v = x_ref[0].reshape(bsz * pairs, 2, wpr, 2 * c)
    ev, od = v[:, 0], v[:, 1]
    ecat_ref[:, 0:wpr, 0:2 * c] = ev
    ocat_ref[:, 0:wpr, 0:2 * c] = od
    ecat_ref[:, 0:wpr - 1, 2 * c:3 * c] = ev[:, 1:wpr, 0:c]
    ocat_ref[:, 0:wpr - 1, 2 * c:3 * c] = od[:, 1:wpr, 0:c]
    lout = bsz * pairs - 1
    acc = None
    for di in range(3):
        src = ecat_ref if di % 2 == 0 else ocat_ref
        a = src[di // 2:di // 2 + lout].reshape(lout * woc, 3 * c)
        z = jnp.dot(a, w_ref[di], preferred_element_type=jnp.float32)
        acc = z if acc is None else acc + z
    y = jnp.maximum(acc + s_ref[...], 0.0).astype(o_ref.dtype)
    y = y.reshape(lout, woc, o_ref.shape[-1])
    for b in range(bsz):
        o_ref[0, b * ho:(b + 1) * ho] = y[b * pairs: b * pairs + ho, 0:wo]


@functools.lru_cache(maxsize=None)
def _c3s2_build(g, bsz, ho, wo, c, cout, pairs, wpr, woc):
    rows = bsz * 2 * pairs
    lout = bsz * pairs - 1
    return pl.pallas_call(
        functools.partial(_c3s2_body, bsz=bsz, ho=ho, wo=wo, c=c,
                          pairs=pairs, wpr=wpr, woc=woc),
        out_shape=jax.ShapeDtypeStruct((g, bsz * ho, wo, cout), jnp.bfloat16),
        grid_spec=pltpu.PrefetchScalarGridSpec(
            num_scalar_prefetch=0,
            grid=(g,),
            in_specs=[
                pl.BlockSpec((1, rows, wpr, 2 * c), lambda i: (i, 0, 0, 0)),
                pl.BlockSpec((3, 3 * c, cout), lambda i: (0, 0, 0)),
                pl.BlockSpec((1, cout), lambda i: (0, 0)),
            ],
            out_specs=pl.BlockSpec((1, bsz * ho, wo, cout), lambda i: (i, 0, 0, 0)),
            scratch_shapes=[pltpu.VMEM((bsz * pairs, woc, 3 * c), jnp.bfloat16),
                            pltpu.VMEM((bsz * pairs, woc, 3 * c), jnp.bfloat16)],
        ),
        compiler_params=pltpu.CompilerParams(
            dimension_semantics=("parallel",),
            vmem_limit_bytes=_VMEM),
        cost_estimate=pl.CostEstimate(
            flops=2 * g * lout * woc * 3 * c * cout * 3, transcendentals=0,
            bytes_accessed=(g * rows * wpr * 2 * c + 9 * c * cout
                            + g * bsz * ho * wo * cout) * 2),
    )


def _conv3_s2(x, wfull, shift):
    n, h, w, c = x.shape
    cout = wfull.shape[1]
    ho, wo = h // 2, w // 2
    pairs = (h + 2) // 2
    wpr = (w + 2 + 1) // 2
    woc = _rup(wo, 16)
    w3 = wfull[:9 * c].reshape(3, 3 * c, cout)
    g = {128: 8, 256: 4, 512: 2}.get(c, 4)
    bsz = n // g
    xp = jnp.pad(x, ((0, 0), (1, 1), (1, 2 * wpr - w - 1), (0, 0)))
    xp = xp.reshape(g, bsz * 2 * pairs, wpr, 2 * c)
    out = _c3s2_build(g, bsz, ho, wo, c, cout, pairs, wpr, woc)(xp, w3, shift)
    return out.reshape(n, ho, wo, cout)


def _ds_body(x_ref, w_ref, s_ref, o_ref, *, ho, wo, c):
    v = x_ref[0].reshape(ho, 2, wo, 2 * c)[:, 0]
    a = v.reshape(ho * wo, 2 * c)
    z = jnp.dot(a, w_ref[...], preferred_element_type=jnp.float32)
    z = z + s_ref[...]
    o_ref[0] = z.astype(o_ref.dtype).reshape(ho, wo, o_ref.shape[-1])


@functools.lru_cache(maxsize=None)
def _ds_build(n, h, wo, c, cout):
    return pl.pallas_call(
        functools.partial(_ds_body, ho=h // 2, wo=wo, c=c),
        out_shape=jax.ShapeDtypeStruct((n, h // 2, wo, cout), jnp.bfloat16),
        grid_spec=pltpu.PrefetchScalarGridSpec(
            num_scalar_prefetch=0,
            grid=(n,),
            in_specs=[
                pl.BlockSpec((1, h, wo, 2 * c), lambda i: (i, 0, 0, 0)),
                pl.BlockSpec((2 * c, cout), lambda i: (0, 0)),
                pl.BlockSpec((1, cout), lambda i: (0, 0)),
            ],
            out_specs=pl.BlockSpec((1, h // 2, wo, cout), lambda i: (i, 0, 0, 0)),
        ),
        compiler_params=pltpu.CompilerParams(
            dimension_semantics=("parallel",),
            vmem_limit_bytes=_VMEM),
        cost_estimate=pl.CostEstimate(
            flops=2 * n * (h // 2) * wo * 2 * c * cout, transcendentals=0,
            bytes_accessed=(n * h * wo * c * 2 + 2 * c * cout
                            + n * (h // 2) * wo * cout) * 2),
    )


def _downsample(x, wfull, shift):
    n, h, w, c = x.shape
    cout = wfull.shape[1]
    wz = jnp.concatenate([wfull, jnp.zeros_like(wfull)], axis=0)
    xv = x.reshape(n, h, w // 2, 2 * c)
    return _ds_build(n, h, w // 2, c, cout)(xv, wz, shift)


def _pool_body(ee_ref, eo_ref, oe_ref, oo_ref, o_ref, *, ho, wo):
    ee, eo, oe, oo = ee_ref[0], eo_ref[0], oe_ref[0], oo_ref[0]
    a = jnp.maximum(jnp.maximum(ee[:, :wo], eo[:, :wo]), ee[:, 1:wo + 1])
    b = jnp.maximum(jnp.maximum(oe[:, :wo], oo[:, :wo]), oe[:, 1:wo + 1])
    o_ref[0] = jnp.maximum(jnp.maximum(a[:ho], b[:ho]), a[1:ho + 1])


@functools.lru_cache(maxsize=None)
def _pool_build(n, he, we, ho, wo, c):
    spec = pl.BlockSpec((1, he, we, c), lambda i: (i, 0, 0, 0))
    return pl.pallas_call(
        functools.partial(_pool_body, ho=ho, wo=wo),
        out_shape=jax.ShapeDtypeStruct((n, ho, wo, c), jnp.bfloat16),
        grid=(n,),
        in_specs=[spec, spec, spec, spec],
        out_specs=pl.BlockSpec((1, ho, wo, c), lambda i: (i, 0, 0, 0)),
        compiler_params=pltpu.CompilerParams(
            dimension_semantics=("parallel",),
            vmem_limit_bytes=_VMEM),
    )


def _maxpool(x):
    n, h, w, c = x.shape
    ho, wo = (h - 1) // 2 + 1, (w - 1) // 2 + 1
    he, we = ho + 1, wo + 1
    xp = jnp.pad(x, ((0, 0), (1, 2 * he - h - 1), (1, 2 * we - w - 1), (0, 0)),
                 constant_values=-jnp.inf)
    ee = xp[:, 0::2, 0::2]
    eo = xp[:, 0::2, 1::2]
    oe = xp[:, 1::2, 0::2]
    oo = xp[:, 1::2, 1::2]
    return _pool_build(n, he, we, ho, wo, c)(ee, eo, oe, oo)


def _tail_body(x_ref, fw_ref, fb_ref, w1_ref, b1_ref, w2_ref, b2_ref,
               w3c_ref, b3c_ref, w3r_ref, b3r_ref, c_ref, r_ref, *, inv_hw, hm):
    x = x_ref[...].astype(jnp.float32)
    pooled = jnp.sum(x, axis=1) * inv_hw
    feats = jnp.dot(pooled.astype(jnp.bfloat16), fw_ref[...],
                    preferred_element_type=jnp.float32) + fb_ref[...]
    h1 = jnp.dot(feats.astype(jnp.bfloat16), w1_ref[...],
                 preferred_element_type=jnp.float32) + b1_ref[...]
    h1 = jnp.maximum(h1, 0.0)
    h2 = jnp.dot(h1.astype(jnp.bfloat16), w2_ref[...],
                 preferred_element_type=jnp.float32) + b2_ref[...]
    h2 = jnp.maximum(h2, 0.0).astype(jnp.bfloat16)
    c_ref[...] = jnp.dot(h2[:, :hm], w3c_ref[...],
                         preferred_element_type=jnp.float32) + b3c_ref[...]
    r_ref[...] = jnp.dot(h2[:, hm:], w3r_ref[...],
                         preferred_element_type=jnp.float32) + b3r_ref[...]


@functools.lru_cache(maxsize=None)
def _tail_build(batch, hw, nc, nr):
    vmem = lambda: pl.BlockSpec(memory_space=pltpu.MemorySpace.VMEM)
    return pl.pallas_call(
        functools.partial(_tail_body, inv_hw=1.0 / hw, hm=32),
        out_shape=(jax.ShapeDtypeStruct((batch, nc), jnp.float32),
                   jax.ShapeDtypeStruct((batch, nr), jnp.float32)),
        in_specs=[vmem() for _ in range(11)],
        out_specs=(vmem(), vmem()),
        compiler_params=pltpu.CompilerParams(vmem_limit_bytes=_VMEM),
    )


def _tail(x, A):
    n, h, w, c = x.shape
    xr = x.reshape(n, h * w, c)
    cw1, cb1 = A["classify_fc1_w"], A["classify_fc1_b"]
    rw1, rb1 = A["regression_fc1_w"], A["regression_fc1_b"]
    cw2, cb2 = A["classify_fc2_w"], A["classify_fc2_b"]
    rw2, rb2 = A["regression_fc2_w"], A["regression_fc2_b"]
    d1 = cw1.shape[1]
    w1 = jnp.concatenate([cw1, rw1], axis=1)
    b1 = jnp.concatenate([cb1, rb1], axis=1)
    z = jnp.zeros_like(cw2)
    w2 = jnp.concatenate(
        [jnp.concatenate([cw2, z], axis=1), jnp.concatenate([z, rw2], axis=1)],
        axis=0)
    b2 = jnp.concatenate([cb2, rb2], axis=1)
    nc = A["classify_fc3_w"].shape[1]
    nr = A["regression_fc3_w"].shape[1]
    return _tail_build(n, h * w, nc, nr)(
        xr, A["fc_w"], A["fc_b"], w1, b1, w2, b2,
        A["classify_fc3_w"], A["classify_fc3_b"],
        A["regression_fc3_w"], A["regression_fc3_b"])


_ARCH = [(64, 3, 1), (128, 4, 2), (256, 6, 2), (512, 3, 2)]


def _forward(A):
    n = A["x"].shape[0]
    x = _conv1_pool(A["x"], A["conv1_w"], A["conv1_shift"])
    h = w = x.shape[1]
    cin = x.shape[3]

    for li, (planes, nblocks, lstride) in enumerate(_ARCH):
        for bi in range(nblocks):
            s = lstride if bi == 0 else 1
            pre = "layer%d_block%d_" % (li, bi)
            hn, wn = h // s, w // s
            if bi > 0:
                x = _bottleneck(x, A[pre + "conv1_w"], A[pre + "conv1_shift"],
                                A[pre + "conv2_w"], A[pre + "conv2_shift"],
                                A[pre + "conv3_w"], A[pre + "conv3_shift"])
                continue
            if bi == 0:
                if s == 2:
                    idm = _downsample(x, A[pre + "downsample_w"],
                                      A[pre + "downsample_shift"])
                    idm = idm.reshape(n * hn * wn, -1)
                else:
                    idm = _mm(x.reshape(n * h * w, cin),
                              A[pre + "downsample_w"],
                              A[pre + "downsample_shift"], relu=False)
            else:
                idm = x.reshape(n * h * w, cin)
            if s == 1:
                y = _bottleneck_front(x, A[pre + "conv1_w"],
                                      A[pre + "conv1_shift"],
                                      A[pre + "conv2_w"],
                                      A[pre + "conv2_shift"])
            else:
                y = _mm(x.reshape(n * h * w, cin),
                        A[pre + "conv1_w"], A[pre + "conv1_shift"], relu=True)
                y = _conv3_s2(y.reshape(n, h, w, planes),
                              A[pre + "conv2_w"], A[pre + "conv2_shift"])
            y = _mm(y.reshape(n * hn * wn, planes),
                    A[pre + "conv3_w"], A[pre + "conv3_shift"],
                    relu=True, res=idm)
            cin = 4 * planes
            h, w = hn, wn
            x = y.reshape(n, h, w, cin)

    return _tail(x, A)


def kernel(
    x,
    conv1_w, conv1_shift,
    layer0_block0_conv1_w, layer0_block0_conv1_shift,
    layer0_block0_conv2_w, layer0_block0_conv2_shift,
    layer0_block0_conv3_w, layer0_block0_conv3_shift,
    layer0_block0_downsample_w, layer0_block0_downsample_shift,
    layer0_block1_conv1_w, layer0_block1_conv1_shift,
    layer0_block1_conv2_w, layer0_block1_conv2_shift,
    layer0_block1_conv3_w, layer0_block1_conv3_shift,
    layer0_block2_conv1_w, layer0_block2_conv1_shift,
    layer0_block2_conv2_w, layer0_block2_conv2_shift,
    layer0_block2_conv3_w, layer0_block2_conv3_shift,
    layer1_block0_conv1_w, layer1_block0_conv1_shift,
    layer1_block0_conv2_w, layer1_block0_conv2_shift,
    layer1_block0_conv3_w, layer1_block0_conv3_shift,
    layer1_block0_downsample_w, layer1_block0_downsample_shift,
    layer1_block1_conv1_w, layer1_block1_conv1_shift,
    layer1_block1_conv2_w, layer1_block1_conv2_shift,
    layer1_block1_conv3_w, layer1_block1_conv3_shift,
    layer1_block2_conv1_w, layer1_block2_conv1_shift,
    layer1_block2_conv2_w, layer1_block2_conv2_shift,
    layer1_block2_conv3_w, layer1_block2_conv3_shift,
    layer1_block3_conv1_w, layer1_block3_conv1_shift,
    layer1_block3_conv2_w, layer1_block3_conv2_shift,
    layer1_block3_conv3_w, layer1_block3_conv3_shift,
    layer2_block0_conv1_w, layer2_block0_conv1_shift,
    layer2_block0_conv2_w, layer2_block0_conv2_shift,
    layer2_block0_conv3_w, layer2_block0_conv3_shift,
    layer2_block0_downsample_w, layer2_block0_downsample_shift,
    layer2_block1_conv1_w, layer2_block1_conv1_shift,
    layer2_block1_conv2_w, layer2_block1_conv2_shift,
    layer2_block1_conv3_w, layer2_block1_conv3_shift,
    layer2_block2_conv1_w, layer2_block2_conv1_shift,
    layer2_block2_conv2_w, layer2_block2_conv2_shift,
    layer2_block2_conv3_w, layer2_block2_conv3_shift,
    layer2_block3_conv1_w, layer2_block3_conv1_shift,
    layer2_block3_conv2_w, layer2_block3_conv2_shift,
    layer2_block3_conv3_w, layer2_block3_conv3_shift,
    layer2_block4_conv1_w, layer2_block4_conv1_shift,
    layer2_block4_conv2_w, layer2_block4_conv2_shift,
    layer2_block4_conv3_w, layer2_block4_conv3_shift,
    layer2_block5_conv1_w, layer2_block5_conv1_shift,
    layer2_block5_conv2_w, layer2_block5_conv2_shift,
    layer2_block5_conv3_w, layer2_block5_conv3_shift,
    layer3_block0_conv1_w, layer3_block0_conv1_shift,
    layer3_block0_conv2_w, layer3_block0_conv2_shift,
    layer3_block0_conv3_w, layer3_block0_conv3_shift,
    layer3_block0_downsample_w, layer3_block0_downsample_shift,
    layer3_block1_conv1_w, layer3_block1_conv1_shift,
    layer3_block1_conv2_w, layer3_block1_conv2_shift,
    layer3_block1_conv3_w, layer3_block1_conv3_shift,
    layer3_block2_conv1_w, layer3_block2_conv1_shift,
    layer3_block2_conv2_w, layer3_block2_conv2_shift,
    layer3_block2_conv3_w, layer3_block2_conv3_shift,
    fc_w, fc_b,
    classify_fc1_w, classify_fc1_b,
    classify_fc2_w, classify_fc2_b,
    classify_fc3_w, classify_fc3_b,
    regression_fc1_w, regression_fc1_b,
    regression_fc2_w, regression_fc2_b,
    regression_fc3_w, regression_fc3_b,
):
    A = dict(locals())
    return _forward(A)
